# baseline (speedup 1.0000x reference)
import functools

import jax
import jax.numpy as jnp
from jax import lax
from jax.experimental import pallas as pl
from jax.experimental.pallas import tpu as pltpu
from jax.experimental.pallas import tpu_sc as plsc

EPS = 1e-5

_NC = 2
_NS = 16
_NW = _NC * _NS

_G = 128
_K = 10


@functools.cache
def _make_sc_gather(total, dim):
    assert total % (_NW * _G) == 0
    per_w = total // _NW
    ngroups = per_w // _G
    assert ngroups % _K == 0
    nsteps = ngroups // _K

    mesh = plsc.VectorSubcoreMesh(core_axis_name="c", subcore_axis_name="s")

    @functools.partial(
        pl.kernel,
        mesh=mesh,
        compiler_params=pltpu.CompilerParams(use_tc_tiling_on_sc=False),
        out_type=jax.ShapeDtypeStruct((total, dim), jnp.float32),
        scratch_types=[
            pltpu.VMEM((ngroups, _G), jnp.int32),
            pltpu.VMEM((_K * _G, dim), jnp.float32),
            pltpu.SemaphoreType.DMA,
        ],
    )
    def gather_kernel(idx_hbm, table_hbm, e_hbm, idx_v, rows_v, sem):
        wid = lax.axis_index("s") * _NC + lax.axis_index("c")
        pltpu.sync_copy(idx_hbm.at[wid], idx_v)

        def step(s, carry):
            base_g = s * _K
            cps = []
            for j in range(_K):
                cps.append(pltpu.async_copy(
                    table_hbm.at[idx_v.at[base_g + j]],
                    rows_v.at[pl.ds(j * _G, _G)],
                    sem))
            for cp in cps:
                cp.wait()
            row0 = wid * per_w + base_g * _G
            pltpu.sync_copy(rows_v, e_hbm.at[pl.ds(row0, _K * _G)])
            return carry

        lax.fori_loop(0, nsteps, step, 0)

    return gather_kernel


@functools.cache
def _make_tc_dist(b, l, dim):
    nb = 256

    def body(e_ref, out_ref):
        e = e_ref[...]
        s = e[:, 0:1, :]
        o = e[:, 1:, :]
        sq = jnp.sum((o - s) ** 2, axis=-1)
        un = jnp.sum(s * s, axis=-1)
        vn = jnp.sum(o * o, axis=-1)
        alpha = jnp.clip(1.0 - un, EPS, 1.0)
        beta = jnp.clip(1.0 - vn, EPS, 1.0)
        x = 1.0 + 2.0 * sq / (alpha * beta)
        x = jnp.maximum(x, 1.0 + EPS)
        out_ref[...] = jnp.log(x + jnp.sqrt((x - 1.0) * (x + 1.0)))

    return pl.pallas_call(
        body,
        grid=(b // nb,),
        in_specs=[pl.BlockSpec((nb, l, dim), lambda i: (i, 0, 0))],
        out_specs=pl.BlockSpec((nb, l - 1), lambda i: (i, 0)),
        out_shape=jax.ShapeDtypeStruct((b, l - 1), jnp.float32),
    )


def kernel(inputs, table):
    b, l = inputs.shape
    n, dim = table.shape
    total = b * l
    idx = inputs.reshape(total).astype(jnp.int32)
    idx3d = idx.reshape(_NW, total // (_NW * _G), _G)
    e_flat = _make_sc_gather(total, dim)(idx3d, table)
    e = e_flat.reshape(b, l, dim)
    dist = _make_tc_dist(b, l, dim)(e)
    return dist, e


# speedup vs baseline: 1.2161x; 1.2161x over previous
import functools

import jax
import jax.numpy as jnp
from jax import lax
from jax.experimental import pallas as pl
from jax.experimental.pallas import tpu as pltpu
from jax.experimental.pallas import tpu_sc as plsc

EPS = 1e-5

_NC = 2
_NS = 16
_NW = _NC * _NS

_G = 128
_K = 5
_RPG = 4


@functools.cache
def _make_sc_gather(total, dim):
    rec_w = dim * _RPG
    per_w = total // _NW
    ngroups = per_w // _G
    assert ngroups % _K == 0
    nsteps = ngroups // _K
    erows_step = _K * _G // _RPG

    mesh = plsc.VectorSubcoreMesh(core_axis_name="c", subcore_axis_name="s")

    @functools.partial(
        pl.kernel,
        mesh=mesh,
        out_type=jax.ShapeDtypeStruct((total // _RPG, rec_w), jnp.float32),
        scratch_types=[
            pltpu.VMEM((ngroups, _G), jnp.int32),
            pltpu.VMEM((ngroups, _G), jnp.int32),
            pltpu.VMEM((_K * _G, rec_w), jnp.float32),
            pltpu.VMEM((erows_step, rec_w), jnp.float32),
            pltpu.SemaphoreType.DMA,
        ],
    )
    def gather_kernel(idx_hbm, table_hbm, e_hbm, idx_v, gidx_v, recs_v,
                      e_v, sem):
        wid = lax.axis_index("s") * _NC + lax.axis_index("c")
        pltpu.sync_copy(idx_hbm.at[wid], idx_v)

        def mk_gidx(g, carry):
            for q in range(_G // 16):
                v = idx_v[g, pl.ds(q * 16, 16)]
                gidx_v[g, pl.ds(q * 16, 16)] = jnp.right_shift(v, 2)
            return carry

        lax.fori_loop(0, ngroups, mk_gidx, 0)

        def step(s, carry):
            base_g = s * _K
            cps = []
            for j in range(_K):
                cps.append(pltpu.async_copy(
                    table_hbm.at[gidx_v.at[base_g + j]],
                    recs_v.at[pl.ds(j * _G, _G)],
                    sem))
            for cp in cps:
                cp.wait()

            def compact(r_out, c2):
                gj = jnp.right_shift(r_out, 3)
                q = jnp.bitwise_and(r_out, (_G // 16) - 1)
                idxv = idx_v[base_g + gj, pl.ds(q * 16, 16)]
                for k in range(16):
                    si = idxv[k]
                    off = jnp.bitwise_and(si, _RPG - 1) * dim
                    row = r_out * 16 + k
                    er = r_out * 4 + (k >> 2)
                    ec = (k & (_RPG - 1)) * dim
                    e_v[er, pl.ds(ec, 16)] = recs_v[row, pl.ds(off, 16)]
                    e_v[er, pl.ds(ec + 16, 16)] = (
                        recs_v[row, pl.ds(off + 16, 16)])
                return c2

            lax.fori_loop(0, _K * _G // 16, compact, 0)
            row0 = wid * (per_w // _RPG) + s * erows_step
            pltpu.sync_copy(e_v, e_hbm.at[pl.ds(row0, erows_step)])
            return carry

        lax.fori_loop(0, nsteps, step, 0)

    return gather_kernel


@functools.cache
def _make_tc_dist(b, l, dim):
    nb = 256

    def body(e_ref, out_ref):
        e = e_ref[...]
        s = e[:, 0:1, :]
        o = e[:, 1:, :]
        sq = jnp.sum((o - s) ** 2, axis=-1)
        un = jnp.sum(s * s, axis=-1)
        vn = jnp.sum(o * o, axis=-1)
        alpha = jnp.clip(1.0 - un, EPS, 1.0)
        beta = jnp.clip(1.0 - vn, EPS, 1.0)
        x = 1.0 + 2.0 * sq / (alpha * beta)
        x = jnp.maximum(x, 1.0 + EPS)
        out_ref[...] = jnp.log(x + jnp.sqrt((x - 1.0) * (x + 1.0)))

    return pl.pallas_call(
        body,
        grid=(b // nb,),
        in_specs=[pl.BlockSpec((nb, l, dim), lambda i: (i, 0, 0))],
        out_specs=pl.BlockSpec((nb, l - 1), lambda i: (i, 0)),
        out_shape=jax.ShapeDtypeStruct((b, l - 1), jnp.float32),
    )


def kernel(inputs, table):
    b, l = inputs.shape
    n, dim = table.shape
    total = b * l
    idx3 = inputs.astype(jnp.int32).reshape(_NW, total // (_NW * _G), _G)
    table2 = table.reshape(n // _RPG, dim * _RPG)
    e2 = _make_sc_gather(total, dim)(idx3, table2)
    e = e2.reshape(b, l, dim)
    dist = _make_tc_dist(b, l, dim)(e)
    return dist, e


# speedup vs baseline: 1.3795x; 1.1344x over previous
import functools

import jax
import jax.numpy as jnp
from jax import lax
from jax.experimental import pallas as pl
from jax.experimental.pallas import tpu as pltpu
from jax.experimental.pallas import tpu_sc as plsc

EPS = 1e-5

_NC = 2
_NS = 16
_NW = _NC * _NS

_PW = 128
_K = 5


@functools.cache
def _make_sc_gather(b, l, dim):
    assert b == _NW * _PW
    assert l % _K == 0
    nsteps = l // _K

    mesh = plsc.VectorSubcoreMesh(core_axis_name="c", subcore_axis_name="s")

    @functools.partial(
        pl.kernel,
        mesh=mesh,
        compiler_params=pltpu.CompilerParams(
            use_tc_tiling_on_sc=False, needs_layout_passes=False),
        out_type=jax.ShapeDtypeStruct((l, dim, b), jnp.float32),
        scratch_types=[
            pltpu.VMEM((l, _PW), jnp.int32),
            pltpu.VMEM((_K * _PW, dim), jnp.float32),
            pltpu.VMEM((dim, _PW), jnp.float32),
            pltpu.SemaphoreType.DMA,
        ],
    )
    def gather_kernel(idx_hbm, table_hbm, e_hbm, idx_v, rows_v, panel_v,
                      sem):
        wid = lax.axis_index("s") * _NC + lax.axis_index("c")
        b0 = wid * _PW
        pltpu.sync_copy(idx_hbm.at[:, pl.ds(b0, _PW)], idx_v)
        lanes = lax.iota(jnp.int32, 16)

        def step(s, carry):
            base_g = s * _K
            cps = []
            for j in range(_K):
                cps.append(pltpu.async_copy(
                    table_hbm.at[idx_v.at[base_g + j]],
                    rows_v.at[pl.ds(j * _PW, _PW)],
                    sem))
            for j in range(_K):
                cps[j].wait()
                for g in range(_PW // 16):
                    rows = j * _PW + g * 16 + lanes
                    for c in range(dim):
                        cols = jnp.full((16,), c, jnp.int32)
                        vals = plsc.load_gather(rows_v, [rows, cols])
                        panel_v[c, pl.ds(g * 16, 16)] = vals
                pltpu.sync_copy(panel_v,
                                e_hbm.at[base_g + j, :, pl.ds(b0, _PW)])
            return carry

        lax.fori_loop(0, nsteps, step, 0)

    return gather_kernel


@functools.cache
def _make_tc_dist(b, l, dim):
    nb = 512

    def body(e_ref, out_ref):
        e = e_ref[...]
        s = e[0:1]
        o = e[1:]
        sq = jnp.sum((o - s) ** 2, axis=1)
        un = jnp.sum(s * s, axis=1)
        vn = jnp.sum(o * o, axis=1)
        alpha = jnp.clip(1.0 - un, EPS, 1.0)
        beta = jnp.clip(1.0 - vn, EPS, 1.0)
        x = 1.0 + 2.0 * sq / (alpha * beta)
        x = jnp.maximum(x, 1.0 + EPS)
        out_ref[...] = jnp.log(x + jnp.sqrt((x - 1.0) * (x + 1.0)))

    return pl.pallas_call(
        body,
        grid=(b // nb,),
        in_specs=[pl.BlockSpec((l, dim, nb), lambda i: (0, 0, i))],
        out_specs=pl.BlockSpec((l - 1, nb), lambda i: (0, i)),
        out_shape=jax.ShapeDtypeStruct((l - 1, b), jnp.float32),
    )


def kernel(inputs, table):
    b, l = inputs.shape
    n, dim = table.shape
    idx_t = jnp.transpose(inputs).astype(jnp.int32)
    e_t = _make_sc_gather(b, l, dim)(idx_t, table)
    e = jnp.transpose(e_t, (2, 0, 1))
    e_v = jnp.transpose(e, (1, 2, 0))
    dist_t = _make_tc_dist(b, l, dim)(e_v)
    dist = jnp.transpose(dist_t)
    return dist, e


# speedup vs baseline: 1.4140x; 1.0250x over previous
import functools

import jax
import jax.numpy as jnp
from jax import lax
from jax.experimental import pallas as pl
from jax.experimental.pallas import tpu as pltpu
from jax.experimental.pallas import tpu_sc as plsc

EPS = 1e-5

_NC = 2
_NS = 16
_NW = _NC * _NS

_PW = 128
_K = 5
_RPG = 4


@functools.cache
def _make_sc_gather(b, l, dim):
    rec_w = dim * _RPG
    assert b == _NW * _PW
    assert l % _K == 0
    nsteps = l // _K

    mesh = plsc.VectorSubcoreMesh(core_axis_name="c", subcore_axis_name="s")

    @functools.partial(
        pl.kernel,
        mesh=mesh,
        compiler_params=pltpu.CompilerParams(needs_layout_passes=False),
        out_type=jax.ShapeDtypeStruct((l, dim, b), jnp.float32),
        scratch_types=[
            pltpu.VMEM((l, _PW), jnp.int32),
            pltpu.VMEM((l, _PW), jnp.int32),
            pltpu.VMEM((_K * _PW, rec_w), jnp.float32),
            pltpu.VMEM((dim, _PW), jnp.float32),
            pltpu.SemaphoreType.DMA,
        ],
    )
    def gather_kernel(idx_hbm, table_hbm, e_hbm, gidx_v, off_v, recs_v,
                      panel_v, sem):
        wid = lax.axis_index("s") * _NC + lax.axis_index("c")
        b0 = wid * _PW
        pltpu.sync_copy(idx_hbm.at[:, pl.ds(b0, _PW)], gidx_v)

        def mk_idx(g, carry):
            for q in range(_PW // 16):
                v = gidx_v[g, pl.ds(q * 16, 16)]
                off_v[g, pl.ds(q * 16, 16)] = (
                    jnp.bitwise_and(v, _RPG - 1) * dim)
                gidx_v[g, pl.ds(q * 16, 16)] = jnp.right_shift(v, 2)
            return carry

        lax.fori_loop(0, l, mk_idx, 0)
        lanes = lax.iota(jnp.int32, 16)

        def step(s, carry):
            base_g = s * _K
            cps = []
            for j in range(_K):
                cps.append(pltpu.async_copy(
                    table_hbm.at[gidx_v.at[base_g + j]],
                    recs_v.at[pl.ds(j * _PW, _PW)],
                    sem))
            for j in range(_K):
                cps[j].wait()
                for g in range(_PW // 16):
                    rows = j * _PW + g * 16 + lanes
                    offs = off_v[base_g + j, pl.ds(g * 16, 16)]
                    for c in range(dim):
                        vals = plsc.load_gather(recs_v, [rows, offs + c])
                        panel_v[c, pl.ds(g * 16, 16)] = vals
                pltpu.sync_copy(panel_v,
                                e_hbm.at[base_g + j, :, pl.ds(b0, _PW)])
            return carry

        lax.fori_loop(0, nsteps, step, 0)

    return gather_kernel


@functools.cache
def _make_tc_dist(b, l, dim):
    nb = 512

    def body(e_ref, out_ref):
        e = e_ref[...]
        s = e[0:1]
        o = e[1:]
        sq = jnp.sum((o - s) ** 2, axis=1)
        un = jnp.sum(s * s, axis=1)
        vn = jnp.sum(o * o, axis=1)
        alpha = jnp.clip(1.0 - un, EPS, 1.0)
        beta = jnp.clip(1.0 - vn, EPS, 1.0)
        x = 1.0 + 2.0 * sq / (alpha * beta)
        x = jnp.maximum(x, 1.0 + EPS)
        out_ref[...] = jnp.log(x + jnp.sqrt((x - 1.0) * (x + 1.0)))

    return pl.pallas_call(
        body,
        grid=(b // nb,),
        in_specs=[pl.BlockSpec((l, dim, nb), lambda i: (0, 0, i))],
        out_specs=pl.BlockSpec((l - 1, nb), lambda i: (0, i)),
        out_shape=jax.ShapeDtypeStruct((l - 1, b), jnp.float32),
    )


def kernel(inputs, table):
    b, l = inputs.shape
    n, dim = table.shape
    idx_t = jnp.transpose(inputs).astype(jnp.int32)
    table2 = table.reshape(n // _RPG, dim * _RPG)
    e_t = _make_sc_gather(b, l, dim)(idx_t, table2)
    e = jnp.transpose(e_t, (2, 0, 1))
    e_v = jnp.transpose(e, (1, 2, 0))
    dist_t = _make_tc_dist(b, l, dim)(e_v)
    dist = jnp.transpose(dist_t)
    return dist, e
